# Initial kernel scaffold; baseline (speedup 1.0000x reference)
import jax
import jax.numpy as jnp
from jax.experimental import pallas as pl


def kernel(position, direction, hash_table, W_d0, b_d0, W_d1, b_d1, W_c0, b_c0, W_c1, b_c1, W_c2, b_c2):
    raise NotImplementedError("write your pallas kernel here")


# speedup vs baseline: 9.6474x; 9.6474x over previous
import functools

import jax
import jax.numpy as jnp
import numpy as np
from jax import lax
from jax.experimental import pallas as pl
from jax.experimental.pallas import tpu as pltpu
from jax.experimental.pallas import tpu_sc as plsc

N_SAMPLES = 131072
NUM_LEVELS = 16
TABLE_SIZE = 524288
FEATURE_DIM = 2
MIN_RES = 16
MAX_RES = 2048
SCENE_BOUND = 1.0

_P1 = np.uint32(2654435761).astype(np.int32)
_P2 = np.uint32(805459861).astype(np.int32)
_MASK = TABLE_SIZE - 1

NC, NS = 2, 16
NW = NC * NS
S_T = N_SAMPLES // NW
C = 512
N_CHUNKS = S_T // C
ROWS_PER_FIRE = 128
FIRES = C * 8 // ROWS_PER_FIRE

_B = np.exp((np.log(MAX_RES) - np.log(MIN_RES)) / (NUM_LEVELS - 1))
_RES = [int(np.floor(MIN_RES * (_B ** l))) for l in range(NUM_LEVELS)]
_CONSTS = np.tile(
    (np.array(_RES, np.float32) * 0.5)[:, None], (1, 16)).astype(np.float32)


def _hash16(px, py, pz, rh):
    sx = px * rh + rh
    sy = py * rh + rh
    sz = pz * rh + rh
    x0 = sx.astype(jnp.int32)
    y0 = sy.astype(jnp.int32)
    z0 = sz.astype(jnp.int32)
    x1 = x0 + 1
    by0 = y0 * _P1
    by1 = by0 + _P1
    bz0 = z0 * _P2
    bz1 = bz0 + _P2
    t00 = by0 ^ bz0
    t01 = by0 ^ bz1
    t10 = by1 ^ bz0
    t11 = by1 ^ bz1
    return [
        (x0 ^ t00) & _MASK, (x0 ^ t01) & _MASK,
        (x0 ^ t10) & _MASK, (x0 ^ t11) & _MASK,
        (x1 ^ t00) & _MASK, (x1 ^ t01) & _MASK,
        (x1 ^ t10) & _MASK, (x1 ^ t11) & _MASK,
    ]


def _encode_body(tbl, pos_t, pos_d, consts, out,
                 pos_c, posd_c, cons_v, idx_a, idx_b, rows_a, rows_b,
                 encbuf, sem_a, sem_b):
    wid = lax.axis_index("s") * NC + lax.axis_index("c")
    base_t = wid * S_T

    pltpu.sync_copy(consts, cons_v)

    iota = lax.iota(jnp.int32, 16)
    pat_row = jnp.right_shift(iota, 1)
    pat_col = iota & 1

    def hashfire(lvl, idxbuf, rowsbuf, sem):
        rh = cons_v[lvl]

        @pl.loop(0, FIRES // 8)
        def _j(j):
            for k in range(8):
                s0 = j * 128 + k * 16
                hs = _hash16(pos_c[0, pl.ds(s0, 16)],
                             pos_c[1, pl.ds(s0, 16)],
                             pos_c[2, pl.ds(s0, 16)], rh)
                for c in range(8):
                    idxbuf[c, j, pl.ds(k * 16, 16)] = hs[c]
            for c in range(8):
                pltpu.async_copy(
                    tbl.at[lvl].at[idxbuf.at[c, j]],
                    rowsbuf.at[pl.ds((c * (FIRES // 8)) * 128 + j * 128,
                                     128)],
                    sem)

    def drain(lvl, idxbuf, rowsbuf, sem):
        for j in range(FIRES // 8):
            for c in range(8):
                pltpu.make_async_copy(
                    tbl.at[lvl].at[idxbuf.at[c, j]],
                    rowsbuf.at[pl.ds((c * (FIRES // 8)) * 128 + j * 128,
                                     128)],
                    sem).wait()

    def interp(lvl, rowsbuf):
        rh = cons_v[lvl]
        col = pat_col + lvl * 2

        @pl.loop(0, C // 8, unroll=4)
        def _s(s):
            do_ = s * 16
            sx = posd_c[0, pl.ds(do_, 16)] * rh + rh
            sy = posd_c[1, pl.ds(do_, 16)] * rh + rh
            sz = posd_c[2, pl.ds(do_, 16)] * rh + rh
            fx = sx - sx.astype(jnp.int32).astype(jnp.float32)
            fy = sy - sy.astype(jnp.int32).astype(jnp.float32)
            fz = sz - sz.astype(jnp.int32).astype(jnp.float32)
            sv0 = s * 8 + pat_row
            v = [plsc.load_gather(rowsbuf, [sv0 + c * C, pat_col])
                 for c in range(8)]
            m00 = v[0] + fx * (v[4] - v[0])
            m01 = v[1] + fx * (v[5] - v[1])
            m10 = v[2] + fx * (v[6] - v[2])
            m11 = v[3] + fx * (v[7] - v[3])
            n0 = m00 + fy * (m10 - m00)
            n1 = m01 + fy * (m11 - m01)
            r = n0 + fz * (n1 - n0)
            row = pat_row + s * 8
            plsc.store_scatter(encbuf, [row, col], r)

    @pl.loop(0, N_CHUNKS)
    def _chunk(ci):
        cb = base_t + ci * C
        for d in range(3):
            pltpu.sync_copy(pos_t.at[d, pl.ds(cb, C)], pos_c.at[d])
            pltpu.sync_copy(pos_d.at[d, pl.ds(2 * cb, 2 * C)], posd_c.at[d])

        hashfire(0, idx_a, rows_a, sem_a)

        @pl.loop(0, NUM_LEVELS // 2)
        def _pair(t):
            la = 2 * t
            lb = 2 * t + 1
            hashfire(lb, idx_b, rows_b, sem_b)
            drain(la, idx_a, rows_a, sem_a)
            interp(la, rows_a)

            @pl.when(t < NUM_LEVELS // 2 - 1)
            def _():
                hashfire(lb + 1, idx_a, rows_a, sem_a)

            drain(lb, idx_b, rows_b, sem_b)
            interp(lb, rows_b)

        pltpu.sync_copy(encbuf, out.at[pl.ds(cb, C)])


def _encode(tbl, pos_t, pos_d, consts):
    mesh = plsc.VectorSubcoreMesh(core_axis_name="c", subcore_axis_name="s",
                                  num_cores=NC, num_subcores=NS)
    f = pl.kernel(
        _encode_body,
        out_type=jax.ShapeDtypeStruct((N_SAMPLES, NUM_LEVELS * FEATURE_DIM),
                                      jnp.float32),
        mesh=mesh,
        scratch_types=[
            pltpu.VMEM((3, C), jnp.float32),
            pltpu.VMEM((3, 2 * C), jnp.float32),
            pltpu.VMEM((NUM_LEVELS, 16), jnp.float32),
            pltpu.VMEM((8, FIRES // 8, ROWS_PER_FIRE), jnp.int32),
            pltpu.VMEM((8, FIRES // 8, ROWS_PER_FIRE), jnp.int32),
            pltpu.VMEM((8 * C, FEATURE_DIM), jnp.float32),
            pltpu.VMEM((8 * C, FEATURE_DIM), jnp.float32),
            pltpu.VMEM((C, NUM_LEVELS * FEATURE_DIM), jnp.float32),
            pltpu.SemaphoreType.DMA,
            pltpu.SemaphoreType.DMA,
        ],
        compiler_params=pltpu.CompilerParams(use_tc_tiling_on_sc=False,
                                             needs_layout_passes=False),
    )
    return f(tbl, pos_t, pos_d, consts)


_SH_C = (0.28209479177387814, 0.48860251190291987, 1.0925484305920792,
         0.94617469575755997, 0.31539156525251999, 0.54627421529603959,
         0.59004358992664352, 2.8906114426405538, 0.45704579946446572,
         0.3731763325901154, 1.4453057213202769)


def _decoder_body(enc, d, wd0, bd0, wd1, bd1, wc0, bc0, wc1, bc1, wc2, bc2,
                  out):
    e = enc[...]
    h = jnp.maximum(jnp.dot(e, wd0[...],
                            preferred_element_type=jnp.float32) + bd0[...],
                    0.0)
    feat16 = jnp.dot(h, wd1[...], preferred_element_type=jnp.float32) \
        + bd1[...]
    density = jnp.exp(jnp.clip(feat16[:, 0:1], -15.0, 15.0))

    x = d[:, 0:1]
    y = d[:, 1:2]
    z = d[:, 2:3]
    xx, yy, zz = x * x, y * y, z * z
    xy, yz, xz = x * y, y * z, x * z
    one = jnp.ones_like(x)
    sh = jnp.concatenate([
        _SH_C[0] * one,
        -_SH_C[1] * y,
        _SH_C[1] * z,
        -_SH_C[1] * x,
        _SH_C[2] * xy,
        -_SH_C[2] * yz,
        _SH_C[3] * zz - _SH_C[4],
        -_SH_C[2] * xz,
        _SH_C[5] * (xx - yy),
        _SH_C[6] * y * (-3.0 * xx + yy),
        _SH_C[7] * xy * z,
        _SH_C[8] * y * (1.0 - 5.0 * zz),
        _SH_C[9] * z * (5.0 * zz - 3.0),
        _SH_C[8] * x * (1.0 - 5.0 * zz),
        _SH_C[10] * z * (xx - yy),
        _SH_C[6] * x * (-xx + 3.0 * yy),
    ], axis=-1)
    xc = jnp.concatenate([feat16, sh], axis=-1)
    hc = jnp.maximum(jnp.dot(xc, wc0[...],
                             preferred_element_type=jnp.float32) + bc0[...],
                     0.0)
    hc = jnp.maximum(jnp.dot(hc, wc1[...],
                             preferred_element_type=jnp.float32) + bc1[...],
                     0.0)
    color = jax.nn.sigmoid(jnp.dot(hc, wc2[...],
                                   preferred_element_type=jnp.float32)
                           + bc2[...])
    out[...] = jnp.concatenate([density, color], axis=-1)


def _decoder(enc, direction, wd0, bd0, wd1, bd1, wc0, bc0, wc1, bc1, wc2,
             bc2):
    bn = 2048
    grid = (N_SAMPLES // bn,)
    full = lambda a: pl.BlockSpec(a.shape, lambda i: (0,) * a.ndim)
    return pl.pallas_call(
        _decoder_body,
        grid=grid,
        in_specs=[
            pl.BlockSpec((bn, enc.shape[1]), lambda i: (i, 0)),
            pl.BlockSpec((bn, 3), lambda i: (i, 0)),
            full(wd0), full(bd0), full(wd1), full(bd1),
            full(wc0), full(bc0), full(wc1), full(bc1),
            full(wc2), full(bc2),
        ],
        out_specs=pl.BlockSpec((bn, 4), lambda i: (i, 0)),
        out_shape=jax.ShapeDtypeStruct((N_SAMPLES, 4), jnp.float32),
        compiler_params=pltpu.CompilerParams(
            dimension_semantics=("parallel",)),
    )(enc, direction, wd0, bd0, wd1, bd1, wc0, bc0, wc1, bc1, wc2, bc2)


def kernel(position, direction, hash_table, W_d0, b_d0, W_d1, b_d1, W_c0,
           b_c0, W_c1, b_c1, W_c2, b_c2):
    pos_t = position.T
    pos_d = jnp.repeat(pos_t, 2, axis=1)
    consts = jnp.asarray(_CONSTS)
    enc = _encode(hash_table, pos_t, pos_d, consts)
    return _decoder(enc, direction, W_d0, b_d0.reshape(1, -1),
                    W_d1, b_d1.reshape(1, -1), W_c0, b_c0.reshape(1, -1),
                    W_c1, b_c1.reshape(1, -1), W_c2, b_c2.reshape(1, -1))


# speedup vs baseline: 9.6592x; 1.0012x over previous
import functools

import jax
import jax.numpy as jnp
import numpy as np
from jax import lax
from jax.experimental import pallas as pl
from jax.experimental.pallas import tpu as pltpu
from jax.experimental.pallas import tpu_sc as plsc

N_SAMPLES = 131072
NUM_LEVELS = 16
TABLE_SIZE = 524288
FEATURE_DIM = 2
MIN_RES = 16
MAX_RES = 2048
SCENE_BOUND = 1.0

_P1 = np.uint32(2654435761).astype(np.int32)
_P2 = np.uint32(805459861).astype(np.int32)
_MASK = TABLE_SIZE - 1

NC, NS = 2, 16
NW = NC * NS
S_T = N_SAMPLES // NW
C = 512
N_CHUNKS = S_T // C
ROWS_PER_FIRE = 128
FIRES = C * 8 // ROWS_PER_FIRE

_B = np.exp((np.log(MAX_RES) - np.log(MIN_RES)) / (NUM_LEVELS - 1))
_RES = [int(np.floor(MIN_RES * (_B ** l))) for l in range(NUM_LEVELS)]
_CONSTS = np.tile(
    (np.array(_RES, np.float32) * 0.5)[:, None], (1, 16)).astype(np.float32)


def _hash16(px, py, pz, rh):
    sx = px * rh + rh
    sy = py * rh + rh
    sz = pz * rh + rh
    x0 = sx.astype(jnp.int32)
    y0 = sy.astype(jnp.int32)
    z0 = sz.astype(jnp.int32)
    x1 = x0 + 1
    by0 = y0 * _P1
    by1 = by0 + _P1
    bz0 = z0 * _P2
    bz1 = bz0 + _P2
    t00 = by0 ^ bz0
    t01 = by0 ^ bz1
    t10 = by1 ^ bz0
    t11 = by1 ^ bz1
    return [
        (x0 ^ t00) & _MASK, (x0 ^ t01) & _MASK,
        (x0 ^ t10) & _MASK, (x0 ^ t11) & _MASK,
        (x1 ^ t00) & _MASK, (x1 ^ t01) & _MASK,
        (x1 ^ t10) & _MASK, (x1 ^ t11) & _MASK,
    ]


def _encode_body(tbl, pos, consts, out,
                 posblk, cons_v, idx_a, idx_b, rows_a, rows_b,
                 encbuf, sem_a, sem_b):
    wid = lax.axis_index("s") * NC + lax.axis_index("c")
    base_t = wid * S_T

    pltpu.sync_copy(consts, cons_v)

    iota = lax.iota(jnp.int32, 16)
    pat_row = jnp.right_shift(iota, 1)
    pat_col = iota & 1
    d0 = jnp.zeros((16,), jnp.int32)
    d1 = d0 + 1
    d2 = d0 + 2

    def hashfire(lvl, idxbuf, rowsbuf, sem):
        rh = cons_v[lvl]

        @pl.loop(0, FIRES // 8)
        def _j(j):
            for k in range(8):
                s0 = j * 128 + k * 16
                sv = s0 + iota
                hs = _hash16(plsc.load_gather(posblk, [sv, d0]),
                             plsc.load_gather(posblk, [sv, d1]),
                             plsc.load_gather(posblk, [sv, d2]), rh)
                for c in range(8):
                    idxbuf[c, j, pl.ds(k * 16, 16)] = hs[c]
            for c in range(8):
                pltpu.async_copy(
                    tbl.at[lvl].at[idxbuf.at[c, j]],
                    rowsbuf.at[pl.ds((c * (FIRES // 8)) * 128 + j * 128,
                                     128)],
                    sem)

    def drain(lvl, idxbuf, rowsbuf, sem):
        for j in range(FIRES // 8):
            for c in range(8):
                pltpu.make_async_copy(
                    tbl.at[lvl].at[idxbuf.at[c, j]],
                    rowsbuf.at[pl.ds((c * (FIRES // 8)) * 128 + j * 128,
                                     128)],
                    sem).wait()

    def interp(lvl, rowsbuf):
        rh = cons_v[lvl]
        col = pat_col + lvl * 2

        @pl.loop(0, C // 8, unroll=4)
        def _s(s):
            sv0 = s * 8 + pat_row
            sx = plsc.load_gather(posblk, [sv0, d0]) * rh + rh
            sy = plsc.load_gather(posblk, [sv0, d1]) * rh + rh
            sz = plsc.load_gather(posblk, [sv0, d2]) * rh + rh
            fx = sx - sx.astype(jnp.int32).astype(jnp.float32)
            fy = sy - sy.astype(jnp.int32).astype(jnp.float32)
            fz = sz - sz.astype(jnp.int32).astype(jnp.float32)
            v = [plsc.load_gather(rowsbuf, [sv0 + c * C, pat_col])
                 for c in range(8)]
            m00 = v[0] + fx * (v[4] - v[0])
            m01 = v[1] + fx * (v[5] - v[1])
            m10 = v[2] + fx * (v[6] - v[2])
            m11 = v[3] + fx * (v[7] - v[3])
            n0 = m00 + fy * (m10 - m00)
            n1 = m01 + fy * (m11 - m01)
            r = n0 + fz * (n1 - n0)
            row = pat_row + s * 8
            plsc.store_scatter(encbuf, [row, col], r)

    @pl.loop(0, N_CHUNKS)
    def _chunk(ci):
        cb = base_t + ci * C
        pltpu.sync_copy(pos.at[pl.ds(cb, C)], posblk)

        hashfire(0, idx_a, rows_a, sem_a)

        @pl.loop(0, NUM_LEVELS // 2)
        def _pair(t):
            la = 2 * t
            lb = 2 * t + 1
            hashfire(lb, idx_b, rows_b, sem_b)
            drain(la, idx_a, rows_a, sem_a)
            interp(la, rows_a)

            @pl.when(t < NUM_LEVELS // 2 - 1)
            def _():
                hashfire(lb + 1, idx_a, rows_a, sem_a)

            drain(lb, idx_b, rows_b, sem_b)
            interp(lb, rows_b)

        pltpu.sync_copy(encbuf, out.at[pl.ds(cb, C)])


def _encode(tbl, pos, consts):
    mesh = plsc.VectorSubcoreMesh(core_axis_name="c", subcore_axis_name="s",
                                  num_cores=NC, num_subcores=NS)
    f = pl.kernel(
        _encode_body,
        out_type=jax.ShapeDtypeStruct((N_SAMPLES, NUM_LEVELS * FEATURE_DIM),
                                      jnp.float32),
        mesh=mesh,
        scratch_types=[
            pltpu.VMEM((C, 3), jnp.float32),
            pltpu.VMEM((NUM_LEVELS, 16), jnp.float32),
            pltpu.VMEM((8, FIRES // 8, ROWS_PER_FIRE), jnp.int32),
            pltpu.VMEM((8, FIRES // 8, ROWS_PER_FIRE), jnp.int32),
            pltpu.VMEM((8 * C, FEATURE_DIM), jnp.float32),
            pltpu.VMEM((8 * C, FEATURE_DIM), jnp.float32),
            pltpu.VMEM((C, NUM_LEVELS * FEATURE_DIM), jnp.float32),
            pltpu.SemaphoreType.DMA,
            pltpu.SemaphoreType.DMA,
        ],
        compiler_params=pltpu.CompilerParams(use_tc_tiling_on_sc=False,
                                             needs_layout_passes=False),
    )
    return f(tbl, pos, consts)


_SH_C = (0.28209479177387814, 0.48860251190291987, 1.0925484305920792,
         0.94617469575755997, 0.31539156525251999, 0.54627421529603959,
         0.59004358992664352, 2.8906114426405538, 0.45704579946446572,
         0.3731763325901154, 1.4453057213202769)


def _decoder_body(enc, d, wd0, bd0, wd1, bd1, wc0, bc0, wc1, bc1, wc2, bc2,
                  out):
    e = enc[...]
    h = jnp.maximum(jnp.dot(e, wd0[...],
                            preferred_element_type=jnp.float32) + bd0[...],
                    0.0)
    feat16 = jnp.dot(h, wd1[...], preferred_element_type=jnp.float32) \
        + bd1[...]
    density = jnp.exp(jnp.clip(feat16[:, 0:1], -15.0, 15.0))

    x = d[:, 0:1]
    y = d[:, 1:2]
    z = d[:, 2:3]
    xx, yy, zz = x * x, y * y, z * z
    xy, yz, xz = x * y, y * z, x * z
    one = jnp.ones_like(x)
    sh = jnp.concatenate([
        _SH_C[0] * one,
        -_SH_C[1] * y,
        _SH_C[1] * z,
        -_SH_C[1] * x,
        _SH_C[2] * xy,
        -_SH_C[2] * yz,
        _SH_C[3] * zz - _SH_C[4],
        -_SH_C[2] * xz,
        _SH_C[5] * (xx - yy),
        _SH_C[6] * y * (-3.0 * xx + yy),
        _SH_C[7] * xy * z,
        _SH_C[8] * y * (1.0 - 5.0 * zz),
        _SH_C[9] * z * (5.0 * zz - 3.0),
        _SH_C[8] * x * (1.0 - 5.0 * zz),
        _SH_C[10] * z * (xx - yy),
        _SH_C[6] * x * (-xx + 3.0 * yy),
    ], axis=-1)
    xc = jnp.concatenate([feat16, sh], axis=-1)
    hc = jnp.maximum(jnp.dot(xc, wc0[...],
                             preferred_element_type=jnp.float32) + bc0[...],
                     0.0)
    hc = jnp.maximum(jnp.dot(hc, wc1[...],
                             preferred_element_type=jnp.float32) + bc1[...],
                     0.0)
    color = jax.nn.sigmoid(jnp.dot(hc, wc2[...],
                                   preferred_element_type=jnp.float32)
                           + bc2[...])
    out[...] = jnp.concatenate([density, color], axis=-1)


def _decoder(enc, direction, wd0, bd0, wd1, bd1, wc0, bc0, wc1, bc1, wc2,
             bc2):
    bn = 2048
    grid = (N_SAMPLES // bn,)
    full = lambda a: pl.BlockSpec(a.shape, lambda i: (0,) * a.ndim)
    return pl.pallas_call(
        _decoder_body,
        grid=grid,
        in_specs=[
            pl.BlockSpec((bn, enc.shape[1]), lambda i: (i, 0)),
            pl.BlockSpec((bn, 3), lambda i: (i, 0)),
            full(wd0), full(bd0), full(wd1), full(bd1),
            full(wc0), full(bc0), full(wc1), full(bc1),
            full(wc2), full(bc2),
        ],
        out_specs=pl.BlockSpec((bn, 4), lambda i: (i, 0)),
        out_shape=jax.ShapeDtypeStruct((N_SAMPLES, 4), jnp.float32),
        compiler_params=pltpu.CompilerParams(
            dimension_semantics=("parallel",)),
    )(enc, direction, wd0, bd0, wd1, bd1, wc0, bc0, wc1, bc1, wc2, bc2)


def kernel(position, direction, hash_table, W_d0, b_d0, W_d1, b_d1, W_c0,
           b_c0, W_c1, b_c1, W_c2, b_c2):
    consts = jnp.asarray(_CONSTS)
    enc = _encode(hash_table, position, consts)
    return _decoder(enc, direction, W_d0, b_d0.reshape(1, -1),
                    W_d1, b_d1.reshape(1, -1), W_c0, b_c0.reshape(1, -1),
                    W_c1, b_c1.reshape(1, -1), W_c2, b_c2.reshape(1, -1))
